# baseline (device time: 68455 ns/iter reference)
import jax
import jax.numpy as jnp
from jax import lax
from jax.experimental import pallas as pl
from jax.experimental.pallas import tpu as pltpu

N_DEV = 8
N_SEG = 8


def kernel(x):
    m_per, n = x.shape
    seg = m_per // N_SEG
    ta = (m_per // 3 + 7) // 8 * 8
    tb = ta
    tc = m_per - ta - tb

    def gray(t):
        t = t % N_DEV
        return jnp.where(t < 4, t, 11 - t)

    def body(x_ref, out_ref, fs, fr, bs, br, ps, pr, cp_sem):
        my_pos = lax.axis_index("i")
        r = gray(my_pos)
        nxt = gray(r + 1)
        prv = gray(r - 1)
        sgn = jnp.where(r % 2 == 1, 1, -1)
        par = gray(r - 3 * sgn)

        barrier_sem = pltpu.get_barrier_semaphore()
        for nbr in (nxt, prv, par):
            pl.semaphore_signal(
                barrier_sem, inc=1,
                device_id=(nbr,), device_id_type=pl.DeviceIdType.MESH,
            )
        pl.semaphore_wait(barrier_sem, 3)

        def desc(src, row0, rows, sems_s, sems_r, i, j, target):
            return pltpu.make_async_remote_copy(
                src_ref=src,
                dst_ref=out_ref.at[pl.ds(row0, rows), :],
                send_sem=sems_s.at[i, j],
                recv_sem=sems_r.at[i, j],
                device_id=(target,),
                device_id_type=pl.DeviceIdType.MESH,
            )

        def own(sems_s, sems_r, s, target):
            return desc(
                x_ref.at[pl.ds(s * seg, seg), :],
                my_pos * m_per + s * seg, seg, sems_s, sems_r, 0, s, target,
            )

        def fwd_chunk(origin_ring, slot, s, sems_s, sems_r, target):
            row0 = gray(origin_ring) * m_per + s * seg
            return desc(
                out_ref.at[pl.ds(row0, seg), :],
                row0, seg, sems_s, sems_r, slot, s, target,
            )

        def third(origin_ring, off, rows, sems_s, sems_r, slot, target):
            row0 = gray(origin_ring) * m_per + off
            return desc(
                out_ref.at[pl.ds(row0, rows), :],
                row0, rows, sems_s, sems_r, slot, 0, target,
            )

        F0 = [own(fs, fr, s, nxt) for s in range(N_SEG)]
        B0 = [own(bs, br, s, prv) for s in range(N_SEG)]
        P0 = [own(ps, pr, s, par) for s in range(N_SEG)]
        F1 = [fwd_chunk(r - 1, 1, s, fs, fr, nxt) for s in range(N_SEG)]
        B1 = [fwd_chunk(r + 1, 1, s, bs, br, prv) for s in range(N_SEG)]
        P2 = [fwd_chunk(r + 2 * sgn, 2, s, ps, pr, par) for s in range(N_SEG)]
        off_f = jnp.where(sgn < 0, 0, ta)
        off_b = ta - off_f
        F2 = third(r - 3, off_f, ta, fs, fr, 2, nxt)
        B2 = third(r + 3, off_b, tb, bs, br, 2, prv)
        P1 = third(r + sgn, ta + tb, tc, ps, pr, 1, par)

        for s in range(N_SEG):
            F0[s].start()
            B0[s].start()
            P0[s].start()
        local = pltpu.make_async_copy(
            x_ref, out_ref.at[pl.ds(my_pos * m_per, m_per), :], cp_sem
        )
        local.start()

        for s in range(N_SEG):
            F0[s].wait_recv()
            F1[s].start()
            B0[s].wait_recv()
            B1[s].start()
        P1.start()

        for s in range(N_SEG):
            F1[s].wait_recv()
            B1[s].wait_recv()
            P2[s].start()

        n_gate = -(-ta // seg)
        for s in range(N_SEG):
            P0[s].wait_recv()
        for s in range(n_gate):
            P2[s].wait_recv()
        F2.start()
        B2.start()
        for s in range(n_gate, N_SEG):
            P2[s].wait_recv()

        F2.wait_recv()
        B2.wait_recv()
        P1.wait_recv()

        local.wait()

        for d in (*F0, *B0, *P0, *F1, *B1, *P2, F2, B2, P1):
            d.wait_send()

    return pl.pallas_call(
        body,
        out_shape=jax.ShapeDtypeStruct((N_DEV * m_per, n), x.dtype),
        in_specs=[pl.BlockSpec(memory_space=pl.ANY)],
        out_specs=pl.BlockSpec(memory_space=pl.ANY),
        scratch_shapes=[
            pltpu.SemaphoreType.DMA((3, N_SEG)),
            pltpu.SemaphoreType.DMA((3, N_SEG)),
            pltpu.SemaphoreType.DMA((3, N_SEG)),
            pltpu.SemaphoreType.DMA((3, N_SEG)),
            pltpu.SemaphoreType.DMA((3, N_SEG)),
            pltpu.SemaphoreType.DMA((3, N_SEG)),
            pltpu.SemaphoreType.DMA,
        ],
        compiler_params=pltpu.CompilerParams(collective_id=0),
    )(x)


# device time: 67709 ns/iter; 1.0110x vs baseline; 1.0110x over previous
import jax
import jax.numpy as jnp
from jax import lax
from jax.experimental import pallas as pl
from jax.experimental.pallas import tpu as pltpu

N_DEV = 8
N_SEG = 4


def kernel(x):
    m_per, n = x.shape
    seg = m_per // N_SEG
    ta = (m_per // 3 + 7) // 8 * 8
    tb = ta
    tc = m_per - ta - tb

    def gray(t):
        t = t % N_DEV
        return jnp.where(t < 4, t, 11 - t)

    def body(x_ref, out_ref, fs, fr, bs, br, ps, pr, cp_sem):
        my_pos = lax.axis_index("i")
        r = gray(my_pos)
        nxt = gray(r + 1)
        prv = gray(r - 1)
        sgn = jnp.where(r % 2 == 1, 1, -1)
        par = gray(r - 3 * sgn)

        barrier_sem = pltpu.get_barrier_semaphore()
        for nbr in (nxt, prv, par):
            pl.semaphore_signal(
                barrier_sem, inc=1,
                device_id=(nbr,), device_id_type=pl.DeviceIdType.MESH,
            )
        pl.semaphore_wait(barrier_sem, 3)

        def desc(src, row0, rows, sems_s, sems_r, i, j, target):
            return pltpu.make_async_remote_copy(
                src_ref=src,
                dst_ref=out_ref.at[pl.ds(row0, rows), :],
                send_sem=sems_s.at[i, j],
                recv_sem=sems_r.at[i, j],
                device_id=(target,),
                device_id_type=pl.DeviceIdType.MESH,
            )

        def own(sems_s, sems_r, s, target):
            return desc(
                x_ref.at[pl.ds(s * seg, seg), :],
                my_pos * m_per + s * seg, seg, sems_s, sems_r, 0, s, target,
            )

        def fwd_chunk(origin_ring, slot, s, sems_s, sems_r, target):
            row0 = gray(origin_ring) * m_per + s * seg
            return desc(
                out_ref.at[pl.ds(row0, seg), :],
                row0, seg, sems_s, sems_r, slot, s, target,
            )

        def third(origin_ring, off, rows, sems_s, sems_r, slot, target):
            row0 = gray(origin_ring) * m_per + off
            return desc(
                out_ref.at[pl.ds(row0, rows), :],
                row0, rows, sems_s, sems_r, slot, 0, target,
            )

        F0 = [own(fs, fr, s, nxt) for s in range(N_SEG)]
        B0 = [own(bs, br, s, prv) for s in range(N_SEG)]
        P0 = [own(ps, pr, s, par) for s in range(N_SEG)]
        F1 = [fwd_chunk(r - 1, 1, s, fs, fr, nxt) for s in range(N_SEG)]
        B1 = [fwd_chunk(r + 1, 1, s, bs, br, prv) for s in range(N_SEG)]
        P2 = [fwd_chunk(r + 2 * sgn, 2, s, ps, pr, par) for s in range(N_SEG)]
        off_f = jnp.where(sgn < 0, 0, ta)
        off_b = ta - off_f
        F2 = third(r - 3, off_f, ta, fs, fr, 2, nxt)
        B2 = third(r + 3, off_b, tb, bs, br, 2, prv)
        P1 = third(r + sgn, ta + tb, tc, ps, pr, 1, par)

        for s in range(N_SEG):
            F0[s].start()
            B0[s].start()
            P0[s].start()
        local = pltpu.make_async_copy(
            x_ref, out_ref.at[pl.ds(my_pos * m_per, m_per), :], cp_sem
        )
        local.start()

        for s in range(N_SEG):
            F0[s].wait_recv()
            F1[s].start()
            B0[s].wait_recv()
            B1[s].start()
        P1.start()

        for s in range(N_SEG):
            F1[s].wait_recv()
            B1[s].wait_recv()
            P2[s].start()

        n_gate = -(-ta // seg)
        for s in range(N_SEG):
            P0[s].wait_recv()
        for s in range(n_gate):
            P2[s].wait_recv()
        F2.start()
        B2.start()
        for s in range(n_gate, N_SEG):
            P2[s].wait_recv()

        F2.wait_recv()
        B2.wait_recv()
        P1.wait_recv()

        local.wait()

        for d in (*F0, *B0, *P0, *F1, *B1, *P2, F2, B2, P1):
            d.wait_send()

    return pl.pallas_call(
        body,
        out_shape=jax.ShapeDtypeStruct((N_DEV * m_per, n), x.dtype),
        in_specs=[pl.BlockSpec(memory_space=pl.ANY)],
        out_specs=pl.BlockSpec(memory_space=pl.ANY),
        scratch_shapes=[
            pltpu.SemaphoreType.DMA((3, N_SEG)),
            pltpu.SemaphoreType.DMA((3, N_SEG)),
            pltpu.SemaphoreType.DMA((3, N_SEG)),
            pltpu.SemaphoreType.DMA((3, N_SEG)),
            pltpu.SemaphoreType.DMA((3, N_SEG)),
            pltpu.SemaphoreType.DMA((3, N_SEG)),
            pltpu.SemaphoreType.DMA,
        ],
        compiler_params=pltpu.CompilerParams(collective_id=0),
    )(x)
